# Initial kernel scaffold; baseline (speedup 1.0000x reference)
import functools

import jax
import jax.numpy as jnp
from jax import lax
from jax.experimental import pallas as pl
from jax.experimental.pallas import tpu as pltpu
from jax.experimental.pallas import tpu_sc as plsc

N = 10000
E = 320000
D = 128
DE = 16
DO = 128

NC = 2
NS = 16
NW = NC * NS
CH = 128
ROWS = E // CH
FULL = ROWS // NW
TAILW = ROWS - FULL * NW
TAILB = FULL * NW * CH
HB = 10240
MAXCH = FULL + 1

_mesh = plsc.VectorSubcoreMesh(core_axis_name="c", subcore_axis_name="s")


def _zero_rows(ref, nrows, width):
    zeros16 = jnp.zeros((16,), jnp.float32)

    def body(i, _):
        for k in range(width // 16):
            ref[i, pl.ds(k * 16, 16)] = zeros16
        return 0

    lax.fori_loop(0, nrows, body, 0)


def _fill_1d(ref, n, value):
    vec = jnp.full((16,), value, jnp.float32)

    def body(i, _):
        ref[pl.ds(i * 16, 16)] = vec
        return 0

    lax.fori_loop(0, n // 16, body, 0)


@functools.partial(
    pl.kernel,
    out_type=jax.ShapeDtypeStruct((NC, HB), jnp.float32),
    mesh=_mesh,
    scratch_types=[
        pltpu.VMEM((MAXCH, CH), jnp.int32),
        pltpu.VMEM((CH,), jnp.float32),
        pltpu.VMEM((HB // NS,), jnp.float32),
        pltpu.VMEM_SHARED((HB,), jnp.float32),
    ],
)
def _sc_outdeg(src2d, out_deg, idx_v, ones_v, zst_v, hist_sh):
    c = lax.axis_index("c")
    s = lax.axis_index("s")
    w = c * NS + s
    stripe = HB // NS

    _fill_1d(ones_v, CH, 1.0)
    _fill_1d(zst_v, stripe, 0.0)
    pltpu.sync_copy(zst_v, hist_sh.at[pl.ds(s * stripe, stripe)])
    plsc.subcore_barrier()

    pltpu.sync_copy(src2d.at[pl.ds(w * FULL, FULL)], idx_v.at[pl.ds(0, FULL)])
    nchunks = FULL + jnp.where(w < TAILW, 1, 0)

    @pl.when(w < TAILW)
    def _():
        pltpu.sync_copy(src2d.at[FULL * NW + w], idx_v.at[FULL])

    def body(j, _):
        pltpu.sync_copy(ones_v, hist_sh.at[idx_v.at[j]], add=True)
        return 0

    lax.fori_loop(0, nchunks, body, 0)
    plsc.subcore_barrier()
    pltpu.sync_copy(hist_sh.at[pl.ds(s * stripe, stripe)],
                    out_deg.at[c, pl.ds(s * stripe, stripe)])


@functools.partial(
    pl.kernel,
    out_type=[
        jax.ShapeDtypeStruct((NC, N, D), jnp.float32),
        jax.ShapeDtypeStruct((NC, N, DE), jnp.float32),
        jax.ShapeDtypeStruct((NC, HB), jnp.float32),
    ],
    mesh=_mesh,
    scratch_types=[
        pltpu.VMEM((MAXCH, CH), jnp.int32),
        pltpu.VMEM((MAXCH, CH), jnp.int32),
        pltpu.VMEM((2, CH, D), jnp.float32),
        pltpu.VMEM((2, CH, DE), jnp.float32),
        pltpu.VMEM((CH, D), jnp.float32),
        pltpu.VMEM((N // NS, DE), jnp.float32),
        pltpu.VMEM((CH,), jnp.float32),
        pltpu.VMEM((HB // NS,), jnp.float32),
        pltpu.VMEM_SHARED((N, D), jnp.float32),
        pltpu.VMEM_SHARED((N, DE), jnp.float32),
        pltpu.VMEM_SHARED((HB,), jnp.float32),
        pltpu.SemaphoreType.DMA,
        pltpu.SemaphoreType.DMA,
        pltpu.SemaphoreType.DMA,
        pltpu.SemaphoreType.DMA,
    ],
)
def _sc_agg(h_hbm, src2d, dst2d, ef_hbm, aggh_out, agge_out, indeg_out,
            src_v, dst_v, rows_v, ef_v, zrow_v, ze_v, ones_v, zst_v,
            aggh_sh, agge_sh, histd_sh, gsem0, gsem1, esem0, esem1):
    c = lax.axis_index("c")
    s = lax.axis_index("s")
    w = c * NS + s
    nstripe = N // NS
    hstripe = HB // NS
    gsems = (gsem0, gsem1)
    esems = (esem0, esem1)

    _zero_rows(zrow_v, CH, D)
    _zero_rows(ze_v, nstripe, DE)
    _fill_1d(ones_v, CH, 1.0)
    _fill_1d(zst_v, hstripe, 0.0)
    base = s * nstripe
    for k in range(nstripe // CH):
        pltpu.sync_copy(zrow_v, aggh_sh.at[pl.ds(base + k * CH, CH)])
    rem = nstripe - (nstripe // CH) * CH
    pltpu.sync_copy(zrow_v.at[pl.ds(0, rem)],
                    aggh_sh.at[pl.ds(base + nstripe - rem, rem)])
    pltpu.sync_copy(ze_v, agge_sh.at[pl.ds(base, nstripe)])
    pltpu.sync_copy(zst_v, histd_sh.at[pl.ds(s * hstripe, hstripe)])
    plsc.subcore_barrier()

    pltpu.sync_copy(src2d.at[pl.ds(w * FULL, FULL)], src_v.at[pl.ds(0, FULL)])
    pltpu.sync_copy(dst2d.at[pl.ds(w * FULL, FULL)], dst_v.at[pl.ds(0, FULL)])
    nchunks = FULL + jnp.where(w < TAILW, 1, 0)

    @pl.when(w < TAILW)
    def _():
        pltpu.sync_copy(src2d.at[FULL * NW + w], src_v.at[FULL])
        pltpu.sync_copy(dst2d.at[FULL * NW + w], dst_v.at[FULL])

    def ebase(j):
        return jnp.where(j < FULL, (w * FULL + j) * CH, TAILB + w * CH)

    def issue(j, b):
        pltpu.async_copy(h_hbm.at[src_v.at[j]], rows_v.at[b], gsems[b])
        pltpu.async_copy(ef_hbm.at[pl.ds(ebase(j), CH)], ef_v.at[b], esems[b])

    issue(0, 0)

    def g_body(g, _):
        for b in range(2):
            j = 2 * g + b
            nb = 1 - b

            @pl.when(j < nchunks)
            def _():
                @pl.when(j + 1 < nchunks)
                def _():
                    issue(j + 1, nb)

                pltpu.make_async_copy(
                    h_hbm.at[src_v.at[j]], rows_v.at[b], gsems[b]).wait()
                pltpu.make_async_copy(
                    ef_hbm.at[pl.ds(0, CH)], ef_v.at[b], esems[b]).wait()
                pltpu.sync_copy(rows_v.at[b], aggh_sh.at[dst_v.at[j]],
                                add=True)
                pltpu.sync_copy(ef_v.at[b], agge_sh.at[dst_v.at[j]],
                                add=True)
                pltpu.sync_copy(ones_v, histd_sh.at[dst_v.at[j]], add=True)
        return 0

    lax.fori_loop(0, (MAXCH + 1) // 2, g_body, 0)
    plsc.subcore_barrier()

    pltpu.sync_copy(aggh_sh.at[pl.ds(base, nstripe)],
                    aggh_out.at[c, pl.ds(base, nstripe)])
    pltpu.sync_copy(agge_sh.at[pl.ds(base, nstripe)],
                    agge_out.at[c, pl.ds(base, nstripe)])
    pltpu.sync_copy(histd_sh.at[pl.ds(s * hstripe, hstripe)],
                    indeg_out.at[c, pl.ds(s * hstripe, hstripe)])


def _prep_body(outdeg_ref, feat_ref, h_ref):
    deg = outdeg_ref[0, :N] + outdeg_ref[1, :N]
    norm = lax.rsqrt(jnp.maximum(deg, 1.0))
    h_ref[...] = feat_ref[...] * norm[:, None]


def _final_body(aggh_ref, agge_ref, wh_ref, we_ref, bias_ref, indeg_ref,
                out_ref):
    aggh = aggh_ref[0] + aggh_ref[1]
    agge = agge_ref[0] + agge_ref[1]
    acc = jnp.dot(aggh, wh_ref[...], preferred_element_type=jnp.float32)
    acc = acc + jnp.dot(agge, we_ref[...], preferred_element_type=jnp.float32)
    indeg = indeg_ref[0, :N] + indeg_ref[1, :N]
    norm = lax.rsqrt(jnp.maximum(indeg, 1.0))
    out_ref[...] = acc * norm[:, None] + bias_ref[...]


@jax.jit
def kernel(feat, edge_index, edge_feat, weight, bias):
    src2d = edge_index[0].reshape(ROWS, CH)
    dst2d = edge_index[1].reshape(ROWS, CH)

    out_deg = _sc_outdeg(src2d)

    h = pl.pallas_call(
        _prep_body,
        out_shape=jax.ShapeDtypeStruct((N, D), jnp.float32),
    )(out_deg, feat)

    aggh_p, agge_p, indeg_p = _sc_agg(h, src2d, dst2d, edge_feat)

    out = pl.pallas_call(
        _final_body,
        out_shape=jax.ShapeDtypeStruct((N, DO), jnp.float32),
    )(aggh_p, agge_p, weight[:D], weight[D:], bias.reshape(1, DO), indeg_p)
    return out


# speedup vs baseline: 7.2651x; 7.2651x over previous
import functools

import jax
import jax.numpy as jnp
from jax import lax
from jax.experimental import pallas as pl
from jax.experimental.pallas import tpu as pltpu
from jax.experimental.pallas import tpu_sc as plsc

N = 10000
E = 320000
D = 128
DE = 16
DO = 128

NC = 2
NS = 16
NW = NC * NS
CH = 128
MAXR = 80
PADE = NW * MAXR * CH
NP = 10240
NSTR = NP // NS

_mesh = plsc.VectorSubcoreMesh(core_axis_name="c", subcore_axis_name="s")


def _nchunks(w):
    return jnp.clip((E - w * MAXR * CH) // CH, 0, MAXR)


def _fill_1d(ref, n, value):
    vec = jnp.full((16,), value, jnp.float32)

    def body(i, _):
        ref[pl.ds(i * 16, 16)] = vec
        return 0

    lax.fori_loop(0, n // 16, body, 0)


@functools.partial(
    pl.kernel,
    out_type=[
        jax.ShapeDtypeStruct((NC, 1, NP), jnp.float32),
        jax.ShapeDtypeStruct((NC, 1, NP), jnp.float32),
        jax.ShapeDtypeStruct((NC, NP, DE), jnp.float32),
    ],
    mesh=_mesh,
    scratch_types=[
        pltpu.VMEM((MAXR, CH), jnp.int32),
        pltpu.VMEM((MAXR, CH), jnp.int32),
        pltpu.VMEM((2, CH, DE), jnp.float32),
        pltpu.VMEM((NSTR, DE), jnp.float32),
        pltpu.VMEM((NSTR,), jnp.float32),
        pltpu.VMEM((CH,), jnp.float32),
        pltpu.VMEM_SHARED((NP,), jnp.float32),
        pltpu.VMEM_SHARED((NP,), jnp.float32),
        pltpu.VMEM_SHARED((NP, DE), jnp.float32),
        pltpu.SemaphoreType.DMA,
        pltpu.SemaphoreType.DMA,
    ],
    compiler_params=pltpu.CompilerParams(use_tc_tiling_on_sc=False),
)
def _sc_edge(src3d, dst3d, ef_hbm, outdeg_out, indeg_out, agge_out,
             src_v, dst_v, ef_v, ze_v, zst_v, ones_v,
             hs_sh, hd_sh, agge_sh, esem0, esem1):
    c = lax.axis_index("c")
    s = lax.axis_index("s")
    w = c * NS + s
    esems = (esem0, esem1)

    def zb(i, _):
        ze_v[i, :] = jnp.zeros((DE,), jnp.float32)
        return 0

    lax.fori_loop(0, NSTR, zb, 0)
    _fill_1d(zst_v, NSTR, 0.0)
    _fill_1d(ones_v, CH, 1.0)
    base = s * NSTR
    pltpu.sync_copy(zst_v, hs_sh.at[pl.ds(base, NSTR)])
    pltpu.sync_copy(zst_v, hd_sh.at[pl.ds(base, NSTR)])
    pltpu.sync_copy(ze_v, agge_sh.at[pl.ds(base, NSTR)])
    plsc.subcore_barrier()

    pltpu.sync_copy(src3d.at[w], src_v)
    pltpu.sync_copy(dst3d.at[w], dst_v)
    nchunks = _nchunks(w)

    def issue(j, b):
        pltpu.async_copy(ef_hbm.at[pl.ds(w * MAXR * CH + j * CH, CH)],
                         ef_v.at[b], esems[b])

    issue(0, 0)

    def g_body(g, _):
        for b in range(2):
            j = 2 * g + b
            nb = 1 - b

            @pl.when(j < nchunks)
            def _():
                @pl.when(j + 1 < nchunks)
                def _():
                    issue(j + 1, nb)

                pltpu.sync_copy(ones_v, hs_sh.at[src_v.at[j]], add=True)
                pltpu.sync_copy(ones_v, hd_sh.at[dst_v.at[j]], add=True)
                pltpu.make_async_copy(
                    ef_hbm.at[pl.ds(0, CH)], ef_v.at[b], esems[b]).wait()
                pltpu.sync_copy(ef_v.at[b], agge_sh.at[dst_v.at[j]],
                                add=True)
        return 0

    lax.fori_loop(0, MAXR // 2, g_body, 0)
    plsc.subcore_barrier()

    pltpu.sync_copy(hs_sh.at[pl.ds(base, NSTR)],
                    outdeg_out.at[c, 0, pl.ds(base, NSTR)])
    pltpu.sync_copy(hd_sh.at[pl.ds(base, NSTR)],
                    indeg_out.at[c, 0, pl.ds(base, NSTR)])
    for k in range(NSTR // CH):
        pltpu.sync_copy(agge_sh.at[pl.ds(base + k * CH, CH)],
                        agge_out.at[c, pl.ds(base + k * CH, CH)])


@functools.partial(
    pl.kernel,
    out_type=jax.ShapeDtypeStruct((NC, NP, D), jnp.float32),
    mesh=_mesh,
    scratch_types=[
        pltpu.VMEM((2, 1, CH), jnp.int32),
        pltpu.VMEM((2, 1, CH), jnp.int32),
        pltpu.VMEM((2, CH, D), jnp.float32),
        pltpu.VMEM_SHARED((NP, D), jnp.float32),
        pltpu.SemaphoreType.DMA,
        pltpu.SemaphoreType.DMA,
        pltpu.SemaphoreType.DMA,
        pltpu.SemaphoreType.DMA,
        pltpu.SemaphoreType.DMA,
        pltpu.SemaphoreType.DMA,
    ],
)
def _sc_aggh(h_hbm, src3d, dst3d, aggh_out,
             sr_v, dr_v, rows_v, aggh_sh, is0, is1, id0, id1, g0, g1):
    c = lax.axis_index("c")
    s = lax.axis_index("s")
    w = c * NS + s
    isems = (is0, is1)
    idsems = (id0, id1)
    gsems = (g0, g1)

    def zb(i, _):
        for k in range(D // 16):
            rows_v[0, i, pl.ds(k * 16, 16)] = jnp.zeros((16,), jnp.float32)
        return 0

    lax.fori_loop(0, CH, zb, 0)
    base = s * NSTR
    for k in range(NSTR // CH):
        pltpu.sync_copy(rows_v.at[0], aggh_sh.at[pl.ds(base + k * CH, CH)])
    plsc.subcore_barrier()

    nchunks = _nchunks(w)

    def issue_idx(j, b):
        pltpu.async_copy(src3d.at[w, pl.ds(j, 1)], sr_v.at[b], isems[b])
        pltpu.async_copy(dst3d.at[w, pl.ds(j, 1)], dr_v.at[b], idsems[b])

    def wait_idx(b):
        pltpu.make_async_copy(
            src3d.at[w, pl.ds(0, 1)], sr_v.at[b], isems[b]).wait()
        pltpu.make_async_copy(
            dst3d.at[w, pl.ds(0, 1)], dr_v.at[b], idsems[b]).wait()

    def issue_g(b):
        pltpu.async_copy(h_hbm.at[sr_v.at[b, 0]], rows_v.at[b], gsems[b])

    def wait_g(b):
        pltpu.make_async_copy(
            h_hbm.at[sr_v.at[b, 0]], rows_v.at[b], gsems[b]).wait()

    issue_idx(0, 0)
    wait_idx(0)
    issue_g(0)

    @pl.when(1 < nchunks)
    def _():
        issue_idx(1, 1)

    def g_body(g, _):
        for b in range(2):
            j = 2 * g + b
            nb = 1 - b

            @pl.when(j < nchunks)
            def _():
                @pl.when(j + 1 < nchunks)
                def _():
                    wait_idx(nb)
                    issue_g(nb)

                wait_g(b)
                pltpu.sync_copy(rows_v.at[b], aggh_sh.at[dr_v.at[b, 0]],
                                add=True)

                @pl.when(j + 2 < nchunks)
                def _():
                    issue_idx(j + 2, b)
        return 0

    lax.fori_loop(0, MAXR // 2, g_body, 0)
    plsc.subcore_barrier()

    for k in range(NSTR // CH):
        pltpu.sync_copy(aggh_sh.at[pl.ds(base + k * CH, CH)],
                        aggh_out.at[c, pl.ds(base + k * CH, CH)])


def _prep_body(outdeg_ref, feat_ref, h_ref):
    deg = outdeg_ref[0, 0, :N] + outdeg_ref[1, 0, :N]
    norm = lax.rsqrt(jnp.maximum(deg, 1.0))
    h_ref[...] = feat_ref[...] * norm[:, None]


def _final_body(aggh_ref, agge_ref, wh_ref, we_ref, bias_ref, indeg_ref,
                out_ref):
    aggh = aggh_ref[0, :N] + aggh_ref[1, :N]
    agge = agge_ref[0, :N] + agge_ref[1, :N]
    acc = jnp.dot(aggh, wh_ref[...], preferred_element_type=jnp.float32)
    acc = acc + jnp.dot(agge, we_ref[...], preferred_element_type=jnp.float32)
    indeg = indeg_ref[0, 0, :N] + indeg_ref[1, 0, :N]
    norm = lax.rsqrt(jnp.maximum(indeg, 1.0))
    out_ref[...] = acc * norm[:, None] + bias_ref[...]


@jax.jit
def kernel(feat, edge_index, edge_feat, weight, bias):
    src3d = jnp.pad(edge_index[0], (0, PADE - E)).reshape(NW, MAXR, CH)
    dst3d = jnp.pad(edge_index[1], (0, PADE - E)).reshape(NW, MAXR, CH)

    outdeg_p, indeg_p, agge_p = _sc_edge(src3d, dst3d, edge_feat)

    h = pl.pallas_call(
        _prep_body,
        out_shape=jax.ShapeDtypeStruct((N, D), jnp.float32),
    )(outdeg_p, feat)

    aggh_p = _sc_aggh(h, src3d, dst3d)

    out = pl.pallas_call(
        _final_body,
        out_shape=jax.ShapeDtypeStruct((N, DO), jnp.float32),
    )(aggh_p, agge_p, weight[:D], weight[D:], bias.reshape(1, DO), indeg_p)
    return out


# speedup vs baseline: 7.4162x; 1.0208x over previous
import functools

import jax
import jax.numpy as jnp
from jax import lax
from jax.experimental import pallas as pl
from jax.experimental.pallas import tpu as pltpu
from jax.experimental.pallas import tpu_sc as plsc

N = 10000
E = 320000
D = 128
DE = 16
DO = 128

NC = 2
NS = 16
NW = NC * NS
CH = 128
MAXR = 80
PADE = NW * MAXR * CH
NP = 10240
NSTR = NP // NS
NPA = 10112
NSTRA = NPA // NS
NBUF = 3

_mesh = plsc.VectorSubcoreMesh(core_axis_name="c", subcore_axis_name="s")


def _nchunks(w):
    return jnp.clip((E - w * MAXR * CH) // CH, 0, MAXR)


def _fill_1d(ref, n, value):
    vec = jnp.full((16,), value, jnp.float32)

    def body(i, _):
        ref[pl.ds(i * 16, 16)] = vec
        return 0

    lax.fori_loop(0, n // 16, body, 0)


@functools.partial(
    pl.kernel,
    out_type=[
        jax.ShapeDtypeStruct((NC, 1, NP), jnp.float32),
        jax.ShapeDtypeStruct((NC, 1, NP), jnp.float32),
        jax.ShapeDtypeStruct((NC, NP, DE), jnp.float32),
    ],
    mesh=_mesh,
    scratch_types=[
        pltpu.VMEM((MAXR, CH), jnp.int32),
        pltpu.VMEM((MAXR, CH), jnp.int32),
        pltpu.VMEM((2, CH, DE), jnp.float32),
        pltpu.VMEM((NSTR, DE), jnp.float32),
        pltpu.VMEM((NSTR,), jnp.float32),
        pltpu.VMEM((CH,), jnp.float32),
        pltpu.VMEM_SHARED((NP,), jnp.float32),
        pltpu.VMEM_SHARED((NP,), jnp.float32),
        pltpu.VMEM_SHARED((NP, DE), jnp.float32),
        pltpu.SemaphoreType.DMA,
        pltpu.SemaphoreType.DMA,
    ],
    compiler_params=pltpu.CompilerParams(use_tc_tiling_on_sc=False),
)
def _sc_edge(src3d, dst3d, ef_hbm, outdeg_out, indeg_out, agge_out,
             src_v, dst_v, ef_v, ze_v, zst_v, ones_v,
             hs_sh, hd_sh, agge_sh, esem0, esem1):
    c = lax.axis_index("c")
    s = lax.axis_index("s")
    w = c * NS + s
    esems = (esem0, esem1)

    def zb(i, _):
        ze_v[i, :] = jnp.zeros((DE,), jnp.float32)
        return 0

    lax.fori_loop(0, NSTR, zb, 0)
    _fill_1d(zst_v, NSTR, 0.0)
    _fill_1d(ones_v, CH, 1.0)
    base = s * NSTR
    pltpu.sync_copy(zst_v, hs_sh.at[pl.ds(base, NSTR)])
    pltpu.sync_copy(zst_v, hd_sh.at[pl.ds(base, NSTR)])
    pltpu.sync_copy(ze_v, agge_sh.at[pl.ds(base, NSTR)])
    plsc.subcore_barrier()

    pltpu.sync_copy(src3d.at[w], src_v)
    pltpu.sync_copy(dst3d.at[w], dst_v)
    nchunks = _nchunks(w)

    def issue(j, b):
        pltpu.async_copy(ef_hbm.at[pl.ds(w * MAXR * CH + j * CH, CH)],
                         ef_v.at[b], esems[b])

    issue(0, 0)

    def g_body(g, _):
        for b in range(2):
            j = 2 * g + b
            nb = 1 - b

            @pl.when(j < nchunks)
            def _():
                @pl.when(j + 1 < nchunks)
                def _():
                    issue(j + 1, nb)

                pltpu.sync_copy(ones_v, hs_sh.at[src_v.at[j]], add=True)
                pltpu.sync_copy(ones_v, hd_sh.at[dst_v.at[j]], add=True)
                pltpu.make_async_copy(
                    ef_hbm.at[pl.ds(0, CH)], ef_v.at[b], esems[b]).wait()
                pltpu.sync_copy(ef_v.at[b], agge_sh.at[dst_v.at[j]],
                                add=True)
        return 0

    lax.fori_loop(0, MAXR // 2, g_body, 0)
    plsc.subcore_barrier()

    pltpu.sync_copy(hs_sh.at[pl.ds(base, NSTR)],
                    outdeg_out.at[c, 0, pl.ds(base, NSTR)])
    pltpu.sync_copy(hd_sh.at[pl.ds(base, NSTR)],
                    indeg_out.at[c, 0, pl.ds(base, NSTR)])
    for k in range(NSTR // CH):
        pltpu.sync_copy(agge_sh.at[pl.ds(base + k * CH, CH)],
                        agge_out.at[c, pl.ds(base + k * CH, CH)])


@functools.partial(
    pl.kernel,
    out_type=jax.ShapeDtypeStruct((NC, NPA, D), jnp.float32),
    mesh=_mesh,
    scratch_types=[
        pltpu.VMEM((NBUF, 1, CH), jnp.int32),
        pltpu.VMEM((NBUF, 1, CH), jnp.int32),
        pltpu.VMEM((NBUF, CH, D), jnp.float32),
        pltpu.VMEM_SHARED((NPA, D), jnp.float32),
        pltpu.SemaphoreType.DMA,
        pltpu.SemaphoreType.DMA,
        pltpu.SemaphoreType.DMA,
        pltpu.SemaphoreType.DMA,
        pltpu.SemaphoreType.DMA,
        pltpu.SemaphoreType.DMA,
        pltpu.SemaphoreType.DMA,
        pltpu.SemaphoreType.DMA,
        pltpu.SemaphoreType.DMA,
    ],
)
def _sc_aggh(h_hbm, src3d, dst3d, aggh_out,
             sr_v, dr_v, rows_v, aggh_sh,
             is0, is1, is2, id0, id1, id2, g0, g1, g2):
    c = lax.axis_index("c")
    s = lax.axis_index("s")
    w = c * NS + s
    isems = (is0, is1, is2)
    idsems = (id0, id1, id2)
    gsems = (g0, g1, g2)

    def zb(i, _):
        for k in range(D // 16):
            rows_v[0, i, pl.ds(k * 16, 16)] = jnp.zeros((16,), jnp.float32)
        return 0

    lax.fori_loop(0, CH, zb, 0)
    base = s * NSTRA
    for k in range(NSTRA // CH):
        pltpu.sync_copy(rows_v.at[0], aggh_sh.at[pl.ds(base + k * CH, CH)])
    rem = NSTRA - (NSTRA // CH) * CH
    pltpu.sync_copy(rows_v.at[0, pl.ds(0, rem)],
                    aggh_sh.at[pl.ds(base + NSTRA - rem, rem)])
    plsc.subcore_barrier()

    nchunks = _nchunks(w)

    def issue_idx(j, b):
        pltpu.async_copy(src3d.at[w, pl.ds(j, 1)], sr_v.at[b], isems[b])
        pltpu.async_copy(dst3d.at[w, pl.ds(j, 1)], dr_v.at[b], idsems[b])

    def wait_idx(b):
        pltpu.make_async_copy(
            src3d.at[w, pl.ds(0, 1)], sr_v.at[b], isems[b]).wait()
        pltpu.make_async_copy(
            dst3d.at[w, pl.ds(0, 1)], dr_v.at[b], idsems[b]).wait()

    def issue_g(b):
        pltpu.async_copy(h_hbm.at[sr_v.at[b, 0]], rows_v.at[b], gsems[b])

    def wait_g(b):
        pltpu.make_async_copy(
            h_hbm.at[sr_v.at[b, 0]], rows_v.at[b], gsems[b]).wait()

    issue_idx(0, 0)
    wait_idx(0)
    issue_g(0)

    @pl.when(1 < nchunks)
    def _():
        issue_idx(1, 1)
        wait_idx(1)
        issue_g(1)

    @pl.when(2 < nchunks)
    def _():
        issue_idx(2, 2)

    def g_body(g, _):
        for b in range(NBUF):
            j = NBUF * g + b
            nxt = (b + 2) % NBUF

            @pl.when(j < nchunks)
            def _():
                @pl.when(j + 2 < nchunks)
                def _():
                    wait_idx(nxt)
                    issue_g(nxt)

                wait_g(b)
                pltpu.sync_copy(rows_v.at[b], aggh_sh.at[dr_v.at[b, 0]],
                                add=True)

                @pl.when(j + NBUF < nchunks)
                def _():
                    issue_idx(j + NBUF, b)
        return 0

    lax.fori_loop(0, (MAXR + NBUF - 1) // NBUF, g_body, 0)
    plsc.subcore_barrier()

    for k in range(NSTRA // CH):
        pltpu.sync_copy(aggh_sh.at[pl.ds(base + k * CH, CH)],
                        aggh_out.at[c, pl.ds(base + k * CH, CH)])
    pltpu.sync_copy(aggh_sh.at[pl.ds(base + NSTRA - rem, rem)],
                    aggh_out.at[c, pl.ds(base + NSTRA - rem, rem)])


def _prep_body(outdeg_ref, feat_ref, h_ref):
    deg = outdeg_ref[0, 0, :N] + outdeg_ref[1, 0, :N]
    norm = lax.rsqrt(jnp.maximum(deg, 1.0))
    h_ref[...] = feat_ref[...] * norm[:, None]


def _final_body(aggh_ref, agge_ref, wh_ref, we_ref, bias_ref, indeg_ref,
                out_ref):
    aggh = aggh_ref[0, :N] + aggh_ref[1, :N]
    agge = agge_ref[0, :N] + agge_ref[1, :N]
    acc = jnp.dot(aggh, wh_ref[...], preferred_element_type=jnp.float32)
    acc = acc + jnp.dot(agge, we_ref[...], preferred_element_type=jnp.float32)
    indeg = indeg_ref[0, 0, :N] + indeg_ref[1, 0, :N]
    norm = lax.rsqrt(jnp.maximum(indeg, 1.0))
    out_ref[...] = acc * norm[:, None] + bias_ref[...]


@jax.jit
def kernel(feat, edge_index, edge_feat, weight, bias):
    src3d = jnp.pad(edge_index[0], (0, PADE - E)).reshape(NW, MAXR, CH)
    dst3d = jnp.pad(edge_index[1], (0, PADE - E)).reshape(NW, MAXR, CH)

    outdeg_p, indeg_p, agge_p = _sc_edge(src3d, dst3d, edge_feat)

    h = pl.pallas_call(
        _prep_body,
        out_shape=jax.ShapeDtypeStruct((N, D), jnp.float32),
    )(outdeg_p, feat)

    aggh_p = _sc_aggh(h, src3d, dst3d)

    out = pl.pallas_call(
        _final_body,
        out_shape=jax.ShapeDtypeStruct((N, DO), jnp.float32),
    )(aggh_p, agge_p, weight[:D], weight[D:], bias.reshape(1, DO), indeg_p)
    return out
